# baseline (speedup 1.0000x reference)
import jax
import jax.numpy as jnp
from jax.experimental import pallas as pl

_POOL = 4
_J_IN = 6
_J_OUT = 24
_BBLK = 8


def _unpool_body(in_ref, out_ref):
    x = in_ref[...]
    Bb, F, J, D = x.shape
    xb = jnp.broadcast_to(
        x[:, :, None, :, None, :], (Bb, F, _POOL, J, _POOL, D)
    )
    r = jax.lax.broadcasted_iota(jnp.int32, (Bb, F, _POOL, J, _POOL, D), 4)
    out_ref[...] = jnp.where(r == 0, xb, 0.0)


def kernel(latent):
    B, F, J, D = latent.shape
    out6 = pl.pallas_call(
        _unpool_body,
        grid=(B // _BBLK,),
        in_specs=[pl.BlockSpec((_BBLK, F, J, D), lambda b: (b, 0, 0, 0))],
        out_specs=pl.BlockSpec(
            (_BBLK, F, _POOL, J, _POOL, D), lambda b: (b, 0, 0, 0, 0, 0)
        ),
        out_shape=jax.ShapeDtypeStruct((B, F, _POOL, J, _POOL, D), latent.dtype),
    )(latent)
    return out6.reshape(B, F * _POOL, _J_OUT, D)


# speedup vs baseline: 1.0088x; 1.0088x over previous
import jax
import jax.numpy as jnp
from jax.experimental import pallas as pl

_POOL = 4
_J_IN = 6
_J_OUT = 24
_BBLK = 4


def _unpool_body(in_ref, out_ref):
    x = in_ref[...]
    Bb, F, J, D = x.shape
    xb = jnp.broadcast_to(
        x[:, :, None, :, None, :], (Bb, F, _POOL, J, _POOL, D)
    )
    r = jax.lax.broadcasted_iota(jnp.int32, (Bb, F, _POOL, J, _POOL, D), 4)
    out_ref[...] = jnp.where(r == 0, xb, 0.0)


def kernel(latent):
    B, F, J, D = latent.shape
    out6 = pl.pallas_call(
        _unpool_body,
        grid=(B // _BBLK,),
        in_specs=[pl.BlockSpec((_BBLK, F, J, D), lambda b: (b, 0, 0, 0))],
        out_specs=pl.BlockSpec(
            (_BBLK, F, _POOL, J, _POOL, D), lambda b: (b, 0, 0, 0, 0, 0)
        ),
        out_shape=jax.ShapeDtypeStruct((B, F, _POOL, J, _POOL, D), latent.dtype),
    )(latent)
    return out6.reshape(B, F * _POOL, _J_OUT, D)
